# Initial kernel scaffold; baseline (speedup 1.0000x reference)
import jax
import jax.numpy as jnp
from jax.experimental import pallas as pl


def kernel(user_fea, W_gender, W_age, W_occupation, W_area):
    raise NotImplementedError("write your pallas kernel here")


# speedup vs baseline: 1.8768x; 1.8768x over previous
import functools

import jax
import jax.numpy as jnp
from jax import lax
from jax.experimental import pallas as pl
from jax.experimental.pallas import tpu as pltpu
from jax.experimental.pallas import tpu_sc as plsc

_B = 16384
_D = 32

_info = plsc.get_sparse_core_info()
_NC = _info.num_cores
_NS = _info.num_subcores
_NW = _NC * _NS
_BPW = _B // _NW
_CH = 256
_NCHUNK = _BPW // _CH

_AGE_OFF = 2
_OCC_OFF = 9


def _emb_body(g_hbm, a_hbm, o_hbm, z_hbm, ws_hbm, wz, out,
              ws_v, izv, izq, igv, iav, iov, rz, rows_v, sem):
    wid = lax.axis_index("s") * _NC + lax.axis_index("c")
    base = wid * _BPW
    pltpu.sync_copy(ws_hbm, ws_v)

    for k in range(_NCHUNK):
        cbase = base + k * _CH
        pltpu.sync_copy(z_hbm.at[pl.ds(cbase, _CH)], izv.at[pl.ds(0, _CH)])

        def shift_body(t, _):
            izq[pl.ds(t * 16, 16)] = izv[pl.ds(t * 16, 16)] >> 2
            return ()

        lax.fori_loop(0, _CH // 16, shift_body, ())
        gather = pltpu.async_copy(wz.at[izq], rz, sem)

        pltpu.sync_copy(g_hbm.at[pl.ds(cbase, _CH)], igv.at[pl.ds(0, _CH)])
        pltpu.sync_copy(a_hbm.at[pl.ds(cbase, _CH)], iav.at[pl.ds(0, _CH)])
        pltpu.sync_copy(o_hbm.at[pl.ds(cbase, _CH)], iov.at[pl.ds(0, _CH)])

        def small_body(i, _):
            for c, (sref, off) in enumerate(
                ((igv, 0), (iav, _AGE_OFF), (iov, _OCC_OFF))
            ):
                s = sref[pl.ds(i, 16)][0] + off
                r = s >> 2
                col = (s & 3) * _D
                rows_v[i, pl.ds(c * _D, 16)] = ws_v[r, pl.ds(col, 16)]
                rows_v[i, pl.ds(c * _D + 16, 16)] = ws_v[r, pl.ds(col + 16, 16)]
            return ()

        lax.fori_loop(0, _CH, small_body, ())
        gather.wait()

        def area_body(i, _):
            col = (izv[pl.ds(i, 16)][0] & 3) * _D
            rows_v[i, pl.ds(3 * _D, 16)] = rz[i, pl.ds(col, 16)]
            rows_v[i, pl.ds(3 * _D + 16, 16)] = rz[i, pl.ds(col + 16, 16)]
            return ()

        lax.fori_loop(0, _CH, area_body, ())
        pltpu.sync_copy(rows_v, out.at[pl.ds(cbase, _CH)])


@jax.jit
def _emb(g, a, o, z, ws, wz):
    mesh = plsc.VectorSubcoreMesh(core_axis_name="c", subcore_axis_name="s")
    f = pl.kernel(
        _emb_body,
        mesh=mesh,
        out_type=jax.ShapeDtypeStruct((_B, 4 * _D), jnp.float32),
        scratch_types=[
            pltpu.VMEM((8, 128), jnp.float32),
            pltpu.VMEM((_CH + 16,), jnp.int32),
            pltpu.VMEM((_CH,), jnp.int32),
            pltpu.VMEM((_CH + 16,), jnp.int32),
            pltpu.VMEM((_CH + 16,), jnp.int32),
            pltpu.VMEM((_CH + 16,), jnp.int32),
            pltpu.VMEM((_CH, 128), jnp.float32),
            pltpu.VMEM((_CH, 128), jnp.float32),
            pltpu.SemaphoreType.DMA,
        ],
    )
    return f(g, a, o, z, ws, wz)


def kernel(user_fea, W_gender, W_age, W_occupation, W_area):
    uf = user_fea.astype(jnp.int32)
    ws = (
        jnp.zeros((32, _D), jnp.float32)
        .at[0:2].set(W_gender)
        .at[_AGE_OFF:_AGE_OFF + 7].set(W_age)
        .at[_OCC_OFF:_OCC_OFF + 21].set(W_occupation)
        .reshape(8, 128)
    )
    wz = W_area.reshape(-1, 128)
    return _emb(uf[:, 0], uf[:, 1], uf[:, 2], uf[:, 3], ws, wz)


# speedup vs baseline: 2.3654x; 1.2603x over previous
import functools

import jax
import jax.numpy as jnp
from jax import lax
from jax.experimental import pallas as pl
from jax.experimental.pallas import tpu as pltpu
from jax.experimental.pallas import tpu_sc as plsc

_B = 16384
_D = 32

_info = plsc.get_sparse_core_info()
_NC = _info.num_cores
_NS = _info.num_subcores
_NW = _NC * _NS
_BPW = _B // _NW
_CH = 256
_NCHUNK = _BPW // _CH

_AGE_OFF = 2
_OCC_OFF = 9


def _emb_body(g_hbm, a_hbm, o_hbm, z_hbm, ws_hbm, wz, out,
              ws_v, igv, iav, iov, izv, izq, rz, rows_v, sem):
    wid = lax.axis_index("s") * _NC + lax.axis_index("c")
    base = wid * _BPW
    pltpu.sync_copy(ws_hbm, ws_v)

    for k in range(_NCHUNK):
        cbase = base + k * _CH
        pltpu.sync_copy(z_hbm.at[pl.ds(cbase, _CH)], izv)

        def zidx_body(t, _):
            izq[pl.ds(t * 16, 16)] = izv[pl.ds(t * 16, 16)] >> 2
            return ()

        lax.fori_loop(0, _CH // 16, zidx_body, (), unroll=4)
        gather = pltpu.async_copy(wz.at[izq], rz, sem)

        pltpu.sync_copy(g_hbm.at[pl.ds(cbase, _CH)], igv)
        pltpu.sync_copy(a_hbm.at[pl.ds(cbase, _CH)], iav)
        pltpu.sync_copy(o_hbm.at[pl.ds(cbase, _CH)], iov)

        def small_body(t, _):
            vg = igv[pl.ds(t * 16, 16)]
            va = iav[pl.ds(t * 16, 16)] + _AGE_OFF
            vo = iov[pl.ds(t * 16, 16)] + _OCC_OFF
            for j in range(16):
                i = t * 16 + j
                for c, s in ((0, vg[j]), (1, va[j]), (2, vo[j])):
                    r = s >> 2
                    col = (s & 3) * _D
                    rows_v[i, pl.ds(c * _D, 16)] = ws_v[r, pl.ds(col, 16)]
                    rows_v[i, pl.ds(c * _D + 16, 16)] = (
                        ws_v[r, pl.ds(col + 16, 16)]
                    )
            return ()

        lax.fori_loop(0, _CH // 16, small_body, ())
        gather.wait()

        def area_body(t, _):
            vc = (izv[pl.ds(t * 16, 16)] & 3) * _D
            for j in range(16):
                i = t * 16 + j
                col = vc[j]
                rows_v[i, pl.ds(3 * _D, 16)] = rz[i, pl.ds(col, 16)]
                rows_v[i, pl.ds(3 * _D + 16, 16)] = rz[i, pl.ds(col + 16, 16)]
            return ()

        lax.fori_loop(0, _CH // 16, area_body, ())
        pltpu.sync_copy(rows_v, out.at[pl.ds(cbase, _CH)])


@jax.jit
def _emb(g, a, o, z, ws, wz):
    mesh = plsc.VectorSubcoreMesh(core_axis_name="c", subcore_axis_name="s")
    f = pl.kernel(
        _emb_body,
        mesh=mesh,
        out_type=jax.ShapeDtypeStruct((_B, 4 * _D), jnp.float32),
        scratch_types=[
            pltpu.VMEM((8, 128), jnp.float32),
            pltpu.VMEM((_CH,), jnp.int32),
            pltpu.VMEM((_CH,), jnp.int32),
            pltpu.VMEM((_CH,), jnp.int32),
            pltpu.VMEM((_CH,), jnp.int32),
            pltpu.VMEM((_CH,), jnp.int32),
            pltpu.VMEM((_CH, 128), jnp.float32),
            pltpu.VMEM((_CH, 128), jnp.float32),
            pltpu.SemaphoreType.DMA,
        ],
    )
    return f(g, a, o, z, ws, wz)


def kernel(user_fea, W_gender, W_age, W_occupation, W_area):
    ufi = user_fea.astype(jnp.int32)
    ws = (
        jnp.zeros((32, _D), jnp.float32)
        .at[0:2].set(W_gender)
        .at[_AGE_OFF:_AGE_OFF + 7].set(W_age)
        .at[_OCC_OFF:_OCC_OFF + 21].set(W_occupation)
        .reshape(8, 128)
    )
    wz = W_area.reshape(-1, 128)
    return _emb(ufi[:, 0], ufi[:, 1], ufi[:, 2], ufi[:, 3], ws, wz)
